# baseline (speedup 1.0000x reference)
import functools

import jax
import jax.numpy as jnp
from jax import lax
from jax.experimental import pallas as pl
from jax.experimental.pallas import tpu as pltpu
from jax.experimental.pallas import tpu_sc as plsc

B, H, W = 32, 1024, 1024
P = 128
L = 16
LANES = 128
SUBL = 8


def kernel(x, probe_x, probe_y):
    n_tile_rows = B * H // SUBL
    xg = (
        x.reshape(n_tile_rows, SUBL, W // LANES, LANES)
        .transpose(0, 2, 1, 3)
        .reshape(B * H * W // LANES, LANES)
    )
    mesh = plsc.VectorSubcoreMesh(core_axis_name="c", subcore_axis_name="s")

    @functools.partial(
        pl.kernel,
        mesh=mesh,
        out_type=jax.ShapeDtypeStruct((B, P), jnp.float32),
        scratch_types=[
            pltpu.VMEM((P,), jnp.int32),
            pltpu.VMEM((P,), jnp.int32),
            pltpu.VMEM((P,), jnp.int32),
            pltpu.VMEM((P, LANES), jnp.float32),
            pltpu.VMEM((P,), jnp.float32),
            pltpu.SemaphoreType.DMA,
        ],
        compiler_params=pltpu.CompilerParams(
            needs_layout_passes=False, skip_device_barrier=True
        ),
    )
    def probe_gather(x_hbm, px_hbm, py_hbm, out_hbm, px_v, py_v, row_v, buf_v,
                     val_v, sem):
        wid = lax.axis_index("s") * 2 + lax.axis_index("c")
        pltpu.sync_copy(px_hbm, px_v)
        pltpu.sync_copy(py_hbm, py_v)
        base = wid * (H * W // LANES)
        for i in range(P // L):
            sl = pl.ds(i * L, L)
            px, py = px_v[sl], py_v[sl]
            row_v[sl] = (
                base
                + (px >> 3) * (SUBL * W // LANES)
                + (py >> 7) * SUBL
                + (px & 7)
            )
        pltpu.async_copy(x_hbm.at[row_v], buf_v, sem).wait()
        lane_ids = lax.iota(jnp.int32, L)
        for j in range(P // L):
            sl = pl.ds(j * L, L)
            val_v[sl] = plsc.load_gather(
                buf_v, [lane_ids + j * L, py_v[sl] & 127]
            )
        pltpu.sync_copy(val_v, out_hbm.at[wid])

    return probe_gather(xg, probe_x, probe_y)


# speedup vs baseline: 1.0617x; 1.0617x over previous
import functools

import jax
import jax.numpy as jnp
from jax import lax
from jax.experimental import pallas as pl
from jax.experimental.pallas import tpu as pltpu
from jax.experimental.pallas import tpu_sc as plsc

B, H, W = 32, 1024, 1024
P = 128
L = 16
LANES = 128
SUBL = 8


def kernel(x, probe_x, probe_y):
    n_tile_rows = B * H // SUBL
    xg = (
        x.reshape(n_tile_rows, SUBL, W // LANES, LANES)
        .transpose(0, 2, 1, 3)
        .reshape(B * H * W // LANES, LANES)
    )
    mesh = plsc.VectorSubcoreMesh(
        core_axis_name="c", subcore_axis_name="s", num_cores=1
    )

    @functools.partial(
        pl.kernel,
        mesh=mesh,
        out_type=jax.ShapeDtypeStruct((B, P), jnp.float32),
        scratch_types=[
            pltpu.VMEM((P,), jnp.int32),
            pltpu.VMEM((P,), jnp.int32),
            pltpu.VMEM((2 * P,), jnp.int32),
            pltpu.VMEM((2 * P, LANES), jnp.float32),
            pltpu.VMEM((2, P), jnp.float32),
            pltpu.SemaphoreType.DMA,
        ],
        compiler_params=pltpu.CompilerParams(
            needs_layout_passes=False, skip_device_barrier=True
        ),
    )
    def probe_gather(x_hbm, px_hbm, py_hbm, out_hbm, px_v, py_v, row_v, buf_v,
                     val_v, sem):
        wid = lax.axis_index("s")
        pltpu.sync_copy(px_hbm, px_v)
        pltpu.sync_copy(py_hbm, py_v)
        for k in range(2):
            base = (2 * wid + k) * (H * W // LANES)
            for i in range(P // L):
                sl = pl.ds(i * L, L)
                px, py = px_v[sl], py_v[sl]
                row_v[pl.ds(k * P + i * L, L)] = (
                    base
                    + (px >> 3) * (SUBL * W // LANES)
                    + (py >> 7) * SUBL
                    + (px & 7)
                )
        pltpu.async_copy(x_hbm.at[row_v], buf_v, sem).wait()
        lane_ids = lax.iota(jnp.int32, L)
        for k in range(2):
            for j in range(P // L):
                sl = pl.ds(j * L, L)
                val_v[k, sl] = plsc.load_gather(
                    buf_v, [lane_ids + k * P + j * L, py_v[sl] & 127]
                )
        pltpu.sync_copy(val_v, out_hbm.at[pl.ds(2 * wid, 2)])

    return probe_gather(xg, probe_x, probe_y)


# speedup vs baseline: 1.0829x; 1.0200x over previous
import functools

import jax
import jax.numpy as jnp
from jax import lax
from jax.experimental import pallas as pl
from jax.experimental.pallas import tpu as pltpu
from jax.experimental.pallas import tpu_sc as plsc

B, H, W = 32, 1024, 1024
P = 128
L = 16
LANES = 128
SUBL = 8


def kernel(x, probe_x, probe_y):
    n_tile_rows = B * H // SUBL
    xg = (
        x.reshape(n_tile_rows, SUBL, W // LANES, LANES)
        .transpose(0, 2, 1, 3)
        .reshape(B * H * W // LANES, LANES)
    )
    mesh = plsc.VectorSubcoreMesh(
        core_axis_name="c", subcore_axis_name="s", num_cores=1
    )

    @functools.partial(
        pl.kernel,
        mesh=mesh,
        out_type=jax.ShapeDtypeStruct((B, P), jnp.float32),
        scratch_types=[
            pltpu.VMEM((P,), jnp.int32),
            pltpu.VMEM((P,), jnp.int32),
            pltpu.VMEM((2 * P,), jnp.int32),
            pltpu.VMEM((2 * P, LANES), jnp.float32),
            pltpu.VMEM((2, P), jnp.float32),
            pltpu.SemaphoreType.DMA,
            pltpu.SemaphoreType.DMA,
            pltpu.SemaphoreType.DMA,
        ],
        compiler_params=pltpu.CompilerParams(
            needs_layout_passes=False, skip_device_barrier=True
        ),
    )
    def probe_gather(x_hbm, px_hbm, py_hbm, out_hbm, px_v, py_v, row_v, buf_v,
                     val_v, sem_px, sem_py, sem_g):
        wid = lax.axis_index("s")
        cp_px = pltpu.async_copy(px_hbm, px_v, sem_px)
        cp_py = pltpu.async_copy(py_hbm, py_v, sem_py)
        cp_px.wait()
        cp_py.wait()
        for k in range(2):
            base = (2 * wid + k) * (H * W // LANES)
            for i in range(P // L):
                sl = pl.ds(i * L, L)
                px, py = px_v[sl], py_v[sl]
                row_v[pl.ds(k * P + i * L, L)] = (
                    base
                    + (px >> 3) * (SUBL * W // LANES)
                    + (py >> 7) * SUBL
                    + (px & 7)
                )
        cp0 = pltpu.async_copy(
            x_hbm.at[row_v.at[pl.ds(0, P)]], buf_v.at[pl.ds(0, P)], sem_g
        )
        cp1 = pltpu.async_copy(
            x_hbm.at[row_v.at[pl.ds(P, P)]], buf_v.at[pl.ds(P, P)], sem_px
        )
        lane_ids = lax.iota(jnp.int32, L)
        cp0.wait()
        for j in range(P // L):
            sl = pl.ds(j * L, L)
            val_v[0, sl] = plsc.load_gather(
                buf_v, [lane_ids + j * L, py_v[sl] & 127]
            )
        cp1.wait()
        for j in range(P // L):
            sl = pl.ds(j * L, L)
            val_v[1, sl] = plsc.load_gather(
                buf_v, [lane_ids + P + j * L, py_v[sl] & 127]
            )
        pltpu.sync_copy(val_v, out_hbm.at[pl.ds(2 * wid, 2)])

    return probe_gather(xg, probe_x, probe_y)


# speedup vs baseline: 1.0831x; 1.0001x over previous
import functools

import jax
import jax.numpy as jnp
from jax import lax
from jax.experimental import pallas as pl
from jax.experimental.pallas import tpu as pltpu
from jax.experimental.pallas import tpu_sc as plsc

B, H, W = 32, 1024, 1024
P = 128
L = 16
LANES = 128
SUBL = 8


def kernel(x, probe_x, probe_y):
    n_tile_rows = B * H // SUBL
    xg = (
        x.reshape(n_tile_rows, SUBL, W // LANES, LANES)
        .transpose(0, 2, 1, 3)
        .reshape(B * H * W // LANES, LANES)
    )
    mesh = plsc.VectorSubcoreMesh(
        core_axis_name="c", subcore_axis_name="s", num_cores=1
    )

    @functools.partial(
        pl.kernel,
        mesh=mesh,
        out_type=jax.ShapeDtypeStruct((B, P), jnp.float32),
        scratch_types=[
            pltpu.VMEM((P,), jnp.int32),
            pltpu.VMEM((P,), jnp.int32),
            pltpu.VMEM((2 * P,), jnp.int32),
            pltpu.VMEM((2 * P, LANES), jnp.float32),
            pltpu.VMEM((2, P), jnp.float32),
            pltpu.SemaphoreType.DMA,
            pltpu.SemaphoreType.DMA,
            pltpu.SemaphoreType.DMA,
        ],
        compiler_params=pltpu.CompilerParams(
            needs_layout_passes=False, skip_device_barrier=True
        ),
    )
    def probe_gather(x_hbm, px_hbm, py_hbm, out_hbm, px_v, py_v, row_v, buf_v,
                     val_v, sem_px, sem_py, sem_g):
        wid = lax.axis_index("s")
        cp_px = pltpu.async_copy(px_hbm, px_v, sem_px)
        cp_py = pltpu.async_copy(py_hbm, py_v, sem_py)
        cp_px.wait()
        cp_py.wait()
        base = 2 * wid * (H * W // LANES)
        for i in range(P // L):
            sl = pl.ds(i * L, L)
            px, py = px_v[sl], py_v[sl]
            row_v[sl] = (
                base
                + (px >> 3) * (SUBL * W // LANES)
                + (py >> 7) * SUBL
                + (px & 7)
            )
        cp0 = pltpu.async_copy(
            x_hbm.at[row_v.at[pl.ds(0, P)]], buf_v.at[pl.ds(0, P)], sem_g
        )
        for i in range(P // L):
            sl = pl.ds(i * L, L)
            row_v[pl.ds(P + i * L, L)] = row_v[sl] + (H * W // LANES)
        cp1 = pltpu.async_copy(
            x_hbm.at[row_v.at[pl.ds(P, P)]], buf_v.at[pl.ds(P, P)], sem_px
        )
        lane_ids = lax.iota(jnp.int32, L)
        cp0.wait()
        for j in range(P // L):
            sl = pl.ds(j * L, L)
            val_v[0, sl] = plsc.load_gather(
                buf_v, [lane_ids + j * L, py_v[sl] & 127]
            )
        cp1.wait()
        for j in range(P // L):
            sl = pl.ds(j * L, L)
            val_v[1, sl] = plsc.load_gather(
                buf_v, [lane_ids + P + j * L, py_v[sl] & 127]
            )
        pltpu.sync_copy(val_v, out_hbm.at[pl.ds(2 * wid, 2)])

    return probe_gather(xg, probe_x, probe_y)


# speedup vs baseline: 1.1556x; 1.0669x over previous
import functools

import jax
import jax.numpy as jnp
from jax import lax
from jax.experimental import pallas as pl
from jax.experimental.pallas import tpu as pltpu
from jax.experimental.pallas import tpu_sc as plsc

B, H, W = 32, 1024, 1024
P = 128
L = 16
LANES = 128
SUBL = 8


def kernel(x, probe_x, probe_y):
    n_tile_rows = B * H // SUBL
    xg = (
        x.reshape(n_tile_rows, SUBL, W // LANES, LANES)
        .transpose(0, 2, 1, 3)
        .reshape(B * H * W)
    )
    mesh = plsc.VectorSubcoreMesh(
        core_axis_name="c", subcore_axis_name="s", num_cores=1
    )

    @functools.partial(
        pl.kernel,
        mesh=mesh,
        out_type=jax.ShapeDtypeStruct((B, P), jnp.float32),
        scratch_types=[
            pltpu.VMEM((P,), jnp.int32),
            pltpu.VMEM((P,), jnp.int32),
            pltpu.VMEM((2 * P,), jnp.int32),
            pltpu.VMEM((2, P), jnp.float32),
            pltpu.SemaphoreType.DMA,
            pltpu.SemaphoreType.DMA,
            pltpu.SemaphoreType.DMA,
        ],
        compiler_params=pltpu.CompilerParams(
            needs_layout_passes=False, skip_device_barrier=True
        ),
    )
    def probe_gather(x_hbm, px_hbm, py_hbm, out_hbm, px_v, py_v, row_v, val_v,
                     sem_px, sem_py, sem_g):
        wid = lax.axis_index("s")
        cp_px = pltpu.async_copy(px_hbm, px_v, sem_px)
        cp_py = pltpu.async_copy(py_hbm, py_v, sem_py)
        cp_px.wait()
        cp_py.wait()
        base = 2 * wid * (H * W)
        for i in range(P // L):
            sl = pl.ds(i * L, L)
            px, py = px_v[sl], py_v[sl]
            row_v[sl] = (
                base
                + (px >> 3) * (SUBL * W)
                + (py >> 7) * (SUBL * LANES)
                + (px & 7) * LANES
                + (py & 127)
            )
        cp0 = pltpu.async_copy(
            x_hbm.at[row_v.at[pl.ds(0, P)]], val_v.at[0], sem_g
        )
        for i in range(P // L):
            sl = pl.ds(i * L, L)
            row_v[pl.ds(P + i * L, L)] = row_v[sl] + (H * W)
        cp1 = pltpu.async_copy(
            x_hbm.at[row_v.at[pl.ds(P, P)]], val_v.at[1], sem_px
        )
        cp0.wait()
        cp1.wait()
        pltpu.sync_copy(val_v, out_hbm.at[pl.ds(2 * wid, 2)])

    return probe_gather(xg, probe_x, probe_y)
